# Initial kernel scaffold; baseline (speedup 1.0000x reference)
import jax
import jax.numpy as jnp
from jax.experimental import pallas as pl


def kernel(token_ids, vocab_table):
    raise NotImplementedError("write your pallas kernel here")


# speedup vs baseline: 151.7738x; 151.7738x over previous
import functools

import jax
import jax.numpy as jnp
from jax import lax
from jax.experimental import pallas as pl
from jax.experimental.pallas import tpu as pltpu
from jax.experimental.pallas import tpu_sc as plsc

_VOCAB = 100000
_NC, _NS, _L = 2, 16, 16
_NW = _NC * _NS
_NCHUNK = 2


def _gather_call(idx_flat, vocab_table, n_flat):
    b_per_w = n_flat // _NW
    chunk = b_per_w // _NCHUNK
    mesh = plsc.VectorSubcoreMesh(core_axis_name="c", subcore_axis_name="s")

    @functools.partial(
        pl.kernel,
        mesh=mesh,
        compiler_params=pltpu.CompilerParams(needs_layout_passes=False),
        out_type=jax.ShapeDtypeStruct((n_flat,), jnp.float32),
        scratch_types=[
            pltpu.VMEM((_VOCAB,), jnp.float32),
            pltpu.VMEM((chunk,), jnp.int32),
            pltpu.VMEM((chunk,), jnp.float32),
        ],
    )
    def k(table_hbm, idx_hbm, out_hbm, table_v, idx_v, out_v):
        wid = lax.axis_index("s") * _NC + lax.axis_index("c")
        base = wid * b_per_w
        pltpu.sync_copy(table_hbm, table_v)
        for c in range(_NCHUNK):
            pltpu.sync_copy(idx_hbm.at[pl.ds(base + c * chunk, chunk)], idx_v)

            def body(i, carry):
                sl = pl.ds(i * _L, _L)
                out_v[sl] = plsc.load_gather(table_v, [idx_v[sl]])
                return carry

            lax.fori_loop(0, chunk // _L, body, 0)
            pltpu.sync_copy(out_v, out_hbm.at[pl.ds(base + c * chunk, chunk)])

    return k(vocab_table, idx_flat)


def kernel(token_ids, vocab_table):
    n_flat = token_ids.size
    out = _gather_call(token_ids.reshape(-1), vocab_table, n_flat)
    return out.reshape(token_ids.shape)


# speedup vs baseline: 166.2245x; 1.0952x over previous
import functools

import jax
import jax.numpy as jnp
from jax import lax
from jax.experimental import pallas as pl
from jax.experimental.pallas import tpu as pltpu
from jax.experimental.pallas import tpu_sc as plsc

_VOCAB = 100000
_NC, _NS, _L = 2, 16, 16
_NW = _NC * _NS
_NCHUNK = 4
_UNROLL = 16


def _gather_call(idx_flat, vocab_table, n_flat):
    b_per_w = n_flat // _NW
    csz = b_per_w // _NCHUNK
    mesh = plsc.VectorSubcoreMesh(core_axis_name="c", subcore_axis_name="s")

    @functools.partial(
        pl.kernel,
        mesh=mesh,
        compiler_params=pltpu.CompilerParams(needs_layout_passes=False),
        out_type=jax.ShapeDtypeStruct((n_flat,), jnp.float32),
        scratch_types=[
            pltpu.VMEM((_VOCAB,), jnp.float32),
            pltpu.VMEM((csz,), jnp.int32),
            pltpu.VMEM((csz,), jnp.int32),
            pltpu.VMEM((csz,), jnp.float32),
            pltpu.VMEM((csz,), jnp.float32),
            pltpu.SemaphoreType.DMA,
            pltpu.SemaphoreType.DMA,
            pltpu.SemaphoreType.DMA,
            pltpu.SemaphoreType.DMA,
            pltpu.SemaphoreType.DMA,
        ],
    )
    def k(table_hbm, idx_hbm, out_hbm, table_v, idx_v0, idx_v1,
          out_v0, out_v1, sem_t, sem_i0, sem_i1, sem_o0, sem_o1):
        idx_b = (idx_v0, idx_v1)
        out_b = (out_v0, out_v1)
        sem_i = (sem_i0, sem_i1)
        sem_o = (sem_o0, sem_o1)
        wid = lax.axis_index("s") * _NC + lax.axis_index("c")
        base = wid * b_per_w

        t_cp = pltpu.async_copy(table_hbm, table_v, sem_t)
        i_cp = [None] * _NCHUNK
        o_cp = [None] * _NCHUNK
        for c in range(min(2, _NCHUNK)):
            i_cp[c] = pltpu.async_copy(
                idx_hbm.at[pl.ds(base + c * csz, csz)], idx_b[c], sem_i[c])
        t_cp.wait()

        for c in range(_NCHUNK):
            i_cp[c].wait()
            if c >= 2:
                o_cp[c - 2].wait()
            src = idx_b[c % 2]
            dst = out_b[c % 2]

            def body(i, carry):
                for u in range(_UNROLL):
                    sl = pl.ds(i * (_L * _UNROLL) + u * _L, _L)
                    dst[sl] = plsc.load_gather(table_v, [src[sl]])
                return carry

            lax.fori_loop(0, csz // (_L * _UNROLL), body, 0)
            o_cp[c] = pltpu.async_copy(
                dst, out_hbm.at[pl.ds(base + c * csz, csz)], sem_o[c % 2])
            if c + 2 < _NCHUNK:
                i_cp[c + 2] = pltpu.async_copy(
                    idx_hbm.at[pl.ds(base + (c + 2) * csz, csz)],
                    idx_b[c % 2], sem_i[c % 2])
        o_cp[_NCHUNK - 2].wait()
        o_cp[_NCHUNK - 1].wait()

    return k(vocab_table, idx_flat)


def kernel(token_ids, vocab_table):
    n_flat = token_ids.size
    out = _gather_call(token_ids.reshape(-1), vocab_table, n_flat)
    return out.reshape(token_ids.shape)


# speedup vs baseline: 210.6091x; 1.2670x over previous
import functools

import jax
import jax.numpy as jnp
from jax import lax
from jax.experimental import pallas as pl
from jax.experimental.pallas import tpu as pltpu
from jax.experimental.pallas import tpu_sc as plsc

_VOCAB = 100000
_NC, _NS, _L = 2, 16, 16
_NW = _NC * _NS
_RCHUNK = 16
_NBUF = 2


def _row_offsets(cols):
    offs = [c for c in range(0, cols - _L + 1, _L)]
    if offs[-1] + _L < cols:
        offs.append(cols - _L)
    return offs


def _gather_call(token_ids, vocab_table):
    rows, cols = token_ids.shape
    r_per_w = rows // _NW
    nchunks = r_per_w // _RCHUNK
    offs = _row_offsets(cols)
    mesh = plsc.VectorSubcoreMesh(core_axis_name="c", subcore_axis_name="s")

    @functools.partial(
        pl.kernel,
        mesh=mesh,
        compiler_params=pltpu.CompilerParams(needs_layout_passes=False),
        out_type=jax.ShapeDtypeStruct((rows, cols), jnp.float32),
        scratch_types=[
            pltpu.VMEM((_VOCAB,), jnp.float32),
            pltpu.VMEM((_RCHUNK, cols), jnp.int32),
            pltpu.VMEM((_RCHUNK, cols), jnp.int32),
            pltpu.VMEM((_RCHUNK, cols), jnp.float32),
            pltpu.VMEM((_RCHUNK, cols), jnp.float32),
            pltpu.SemaphoreType.DMA,
            pltpu.SemaphoreType.DMA,
            pltpu.SemaphoreType.DMA,
            pltpu.SemaphoreType.DMA,
            pltpu.SemaphoreType.DMA,
        ],
    )
    def k(table_hbm, idx_hbm, out_hbm, table_v, idx_v0, idx_v1,
          out_v0, out_v1, sem_t, sem_i0, sem_i1, sem_o0, sem_o1):
        idx_b = (idx_v0, idx_v1)
        out_b = (out_v0, out_v1)
        sem_i = (sem_i0, sem_i1)
        sem_o = (sem_o0, sem_o1)
        wid = lax.axis_index("s") * _NC + lax.axis_index("c")
        base = wid * r_per_w

        t_cp = pltpu.async_copy(table_hbm, table_v, sem_t)
        i_cp = [None] * nchunks
        o_cp = [None] * nchunks
        for c in range(min(_NBUF, nchunks)):
            i_cp[c] = pltpu.async_copy(
                idx_hbm.at[pl.ds(base + c * _RCHUNK, _RCHUNK)],
                idx_b[c % _NBUF], sem_i[c % _NBUF])
        t_cp.wait()

        for c in range(nchunks):
            i_cp[c].wait()
            if c >= _NBUF:
                o_cp[c - _NBUF].wait()
            src = idx_b[c % _NBUF]
            dst = out_b[c % _NBUF]

            def body(r, carry):
                idxs = [src[r, pl.ds(o, _L)] for o in offs]
                vals = [plsc.load_gather(table_v, [ix]) for ix in idxs]
                for o, v in zip(offs, vals):
                    dst[r, pl.ds(o, _L)] = v
                return carry

            lax.fori_loop(0, _RCHUNK, body, 0)
            o_cp[c] = pltpu.async_copy(
                dst, out_hbm.at[pl.ds(base + c * _RCHUNK, _RCHUNK)],
                sem_o[c % _NBUF])
            if c + _NBUF < nchunks:
                i_cp[c + _NBUF] = pltpu.async_copy(
                    idx_hbm.at[pl.ds(base + (c + _NBUF) * _RCHUNK, _RCHUNK)],
                    idx_b[c % _NBUF], sem_i[c % _NBUF])
        o_cp[nchunks - 2].wait()
        o_cp[nchunks - 1].wait()

    return k(vocab_table, token_ids)


def kernel(token_ids, vocab_table):
    return _gather_call(token_ids, vocab_table)


# speedup vs baseline: 270.5554x; 1.2846x over previous
import functools

import jax
import jax.numpy as jnp
from jax import lax
from jax.experimental import pallas as pl
from jax.experimental.pallas import tpu as pltpu
from jax.experimental.pallas import tpu_sc as plsc

_VOCAB = 100000
_NC, _NS, _L = 2, 16, 16
_NW = _NC * _NS
_RCHUNK = 40
_NBUF = 2


def _gather_call(idx_t, vocab_table):
    rows, cols = idx_t.shape
    cb = cols // _NW
    nchunks = rows // _RCHUNK
    rpair = _RCHUNK // 2
    nv = cb // _L
    mesh = plsc.VectorSubcoreMesh(core_axis_name="c", subcore_axis_name="s")

    @functools.partial(
        pl.kernel,
        mesh=mesh,
        compiler_params=pltpu.CompilerParams(needs_layout_passes=False),
        out_type=jax.ShapeDtypeStruct((rows, cols), jnp.float32),
        scratch_types=[
            pltpu.VMEM((_VOCAB,), jnp.float32),
            pltpu.VMEM((_RCHUNK, cb), jnp.int32),
            pltpu.VMEM((_RCHUNK, cb), jnp.int32),
            pltpu.VMEM((_RCHUNK, cb), jnp.float32),
            pltpu.VMEM((_RCHUNK, cb), jnp.float32),
            pltpu.SemaphoreType.DMA,
            pltpu.SemaphoreType.DMA,
            pltpu.SemaphoreType.DMA,
            pltpu.SemaphoreType.DMA,
            pltpu.SemaphoreType.DMA,
        ],
    )
    def k(table_hbm, idx_hbm, out_hbm, table_v, idx_v0, idx_v1,
          out_v0, out_v1, sem_t, sem_i0, sem_i1, sem_o0, sem_o1):
        idx_b = (idx_v0, idx_v1)
        out_b = (out_v0, out_v1)
        sem_i = (sem_i0, sem_i1)
        sem_o = (sem_o0, sem_o1)
        wid = lax.axis_index("s") * _NC + lax.axis_index("c")
        col0 = wid * cb

        t_cp = pltpu.async_copy(table_hbm, table_v, sem_t)
        i_cp = [None] * nchunks
        o_cp = [None] * nchunks
        for c in range(min(_NBUF, nchunks)):
            i_cp[c] = pltpu.async_copy(
                idx_hbm.at[pl.ds(c * _RCHUNK, _RCHUNK), pl.ds(col0, cb)],
                idx_b[c % _NBUF], sem_i[c % _NBUF])
        t_cp.wait()

        for c in range(nchunks):
            i_cp[c].wait()
            if c >= _NBUF:
                o_cp[c - _NBUF].wait()
            src = idx_b[c % _NBUF]
            dst = out_b[c % _NBUF]

            def body(r, carry):
                locs = [(r * 2 + j, pl.ds(v * _L, _L))
                        for j in range(2) for v in range(nv)]
                idxs = [src[rr, sl] for rr, sl in locs]
                vals = [plsc.load_gather(table_v, [ix]) for ix in idxs]
                for (rr, sl), v in zip(locs, vals):
                    dst[rr, sl] = v
                return carry

            lax.fori_loop(0, rpair, body, 0)
            o_cp[c] = pltpu.async_copy(
                dst, out_hbm.at[pl.ds(c * _RCHUNK, _RCHUNK), pl.ds(col0, cb)],
                sem_o[c % _NBUF])
            if c + _NBUF < nchunks:
                i_cp[c + _NBUF] = pltpu.async_copy(
                    idx_hbm.at[pl.ds((c + _NBUF) * _RCHUNK, _RCHUNK),
                               pl.ds(col0, cb)],
                    idx_b[c % _NBUF], sem_i[c % _NBUF])
        o_cp[nchunks - 2].wait()
        o_cp[nchunks - 1].wait()

    return k(vocab_table, idx_t)


def kernel(token_ids, vocab_table):
    out_t = _gather_call(token_ids.T, vocab_table)
    return out_t.T


# speedup vs baseline: 276.6531x; 1.0225x over previous
import functools

import jax
import jax.numpy as jnp
from jax import lax
from jax.experimental import pallas as pl
from jax.experimental.pallas import tpu as pltpu
from jax.experimental.pallas import tpu_sc as plsc

_VOCAB = 100000
_NC, _NS, _L = 2, 16, 16
_NW = _NC * _NS
_RCHUNK = 40
_NBUF = 2


def _gather_call(idx_t, vocab_table):
    rows, cols = idx_t.shape
    cb = cols // _NW
    nchunks = rows // _RCHUNK
    rpair = _RCHUNK // 2
    nv = cb // _L
    mesh = plsc.VectorSubcoreMesh(core_axis_name="c", subcore_axis_name="s")

    @functools.partial(
        pl.kernel,
        mesh=mesh,
        compiler_params=pltpu.CompilerParams(needs_layout_passes=False),
        out_type=jax.ShapeDtypeStruct((rows, cols), jnp.float32),
        scratch_types=[
            pltpu.VMEM((_VOCAB,), jnp.float32),
            pltpu.VMEM((_RCHUNK, cb), jnp.int32),
            pltpu.VMEM((_RCHUNK, cb), jnp.int32),
            pltpu.VMEM((_RCHUNK, cb), jnp.int32),
            pltpu.VMEM((_RCHUNK, cb), jnp.int32),
            pltpu.VMEM((_RCHUNK, cb), jnp.float32),
            pltpu.VMEM((_RCHUNK, cb), jnp.float32),
            pltpu.SemaphoreType.DMA,
            pltpu.SemaphoreType.DMA,
            pltpu.SemaphoreType.DMA,
            pltpu.SemaphoreType.DMA,
            pltpu.SemaphoreType.DMA,
            pltpu.SemaphoreType.DMA,
            pltpu.SemaphoreType.DMA,
        ],
    )
    def k(table_hbm, idx_hbm, out_hbm, table_v, idx_v0, idx_v1, idx_v2,
          idx_v3, out_v0, out_v1, sem_t, sem_i0, sem_i1, sem_i2, sem_i3,
          sem_o0, sem_o1):
        idx_b = (idx_v0, idx_v1, idx_v2, idx_v3)
        out_b = (out_v0, out_v1)
        sem_i = (sem_i0, sem_i1, sem_i2, sem_i3)
        sem_o = (sem_o0, sem_o1)
        nib = len(idx_b)
        wid = lax.axis_index("s") * _NC + lax.axis_index("c")
        col0 = wid * cb

        i_cp = [None] * nchunks
        o_cp = [None] * nchunks
        for c in range(min(nib, nchunks)):
            i_cp[c] = pltpu.async_copy(
                idx_hbm.at[pl.ds(c * _RCHUNK, _RCHUNK), pl.ds(col0, cb)],
                idx_b[c % nib], sem_i[c % nib])
        t_cp = pltpu.async_copy(table_hbm, table_v, sem_t)
        t_cp.wait()

        for c in range(nchunks):
            i_cp[c].wait()
            if c >= _NBUF:
                o_cp[c - _NBUF].wait()
            src = idx_b[c % nib]
            dst = out_b[c % _NBUF]

            def body(r, carry):
                locs = [(r * 2 + j, pl.ds(v * _L, _L))
                        for j in range(2) for v in range(nv)]
                idxs = [src[rr, sl] for rr, sl in locs]
                vals = [plsc.load_gather(table_v, [ix]) for ix in idxs]
                for (rr, sl), v in zip(locs, vals):
                    dst[rr, sl] = v
                return carry

            lax.fori_loop(0, rpair, body, 0)
            o_cp[c] = pltpu.async_copy(
                dst, out_hbm.at[pl.ds(c * _RCHUNK, _RCHUNK), pl.ds(col0, cb)],
                sem_o[c % _NBUF])
            if c + nib < nchunks:
                i_cp[c + nib] = pltpu.async_copy(
                    idx_hbm.at[pl.ds((c + nib) * _RCHUNK, _RCHUNK),
                               pl.ds(col0, cb)],
                    idx_b[c % nib], sem_i[c % nib])
        o_cp[nchunks - 2].wait()
        o_cp[nchunks - 1].wait()

    return k(vocab_table, idx_t)


def kernel(token_ids, vocab_table):
    out_t = _gather_call(token_ids.T, vocab_table)
    return out_t.T
